# Initial kernel scaffold; baseline (speedup 1.0000x reference)
import jax
import jax.numpy as jnp
from jax.experimental import pallas as pl


def kernel(pred_logits, pred_boxes, gt_boxes, gt_labels, image_size):
    raise NotImplementedError("write your pallas kernel here")


# speedup vs baseline: 12.8740x; 12.8740x over previous
import jax
import jax.numpy as jnp
from jax.experimental import pallas as pl

B, Q, C, G = 16, 4096, 80, 100
OTA_K = 5
ALPHA, GAMMA = 0.25, 2.0
CLASS_W, BBOX_W, GIOU_W = 1.0, 5.0, 2.0
CENTER_RADIUS = 2.5
BIG = 1e30


def _match_kernel(logits_ref, boxes_ref, gt_t_ref, labels_ref, img_ref,
                  cost_ref, matching_ref, qidx_ref):
    logits = logits_ref[0]
    boxes = boxes_ref[0]
    gt_t = gt_t_ref[0]
    labels = labels_ref[0]
    img = img_ref[0]

    rowi = jax.lax.broadcasted_iota(jnp.int32, (Q, G), 0)
    coli = jax.lax.broadcasted_iota(jnp.int32, (Q, G), 1)

    bx0 = boxes[:, 0:1]
    by0 = boxes[:, 1:2]
    bx1 = boxes[:, 2:3]
    by1 = boxes[:, 3:4]
    gx0 = gt_t[0:1, :]
    gy0 = gt_t[1:2, :]
    gx1 = gt_t[2:3, :]
    gy1 = gt_t[3:4, :]

    i0 = img[0:1, 0:1]
    i1 = img[0:1, 1:2]
    i2 = img[0:1, 2:3]
    i3 = img[0:1, 3:4]
    cost_bbox = (jnp.abs(bx0 / i0 - gx0 / i0)
                 + jnp.abs(by0 / i1 - gy0 / i1)
                 + jnp.abs(bx1 / i2 - gx1 / i2)
                 + jnp.abs(by1 / i3 - gy1 / i3))

    area_a = (bx1 - bx0) * (by1 - by0)
    area_b = (gx1 - gx0) * (gy1 - gy0)
    w = jnp.maximum(jnp.minimum(bx1, gx1) - jnp.maximum(bx0, gx0), 0.0)
    h = jnp.maximum(jnp.minimum(by1, gy1) - jnp.maximum(by0, gy0), 0.0)
    inter = w * h
    union = (area_a + area_b) - inter
    iou = inter / (union + 1e-8)
    w2 = jnp.maximum(jnp.maximum(bx1, gx1) - jnp.minimum(bx0, gx0), 0.0)
    h2 = jnp.maximum(jnp.maximum(by1, gy1) - jnp.minimum(by0, gy0), 0.0)
    area_c = w2 * h2
    giou = iou - (area_c - union) / (area_c + 1e-8)

    acx = (bx0 + bx1) / 2
    acy = (by0 + by1) / 2
    gcx = (gx0 + gx1) / 2
    gcy = (gy0 + gy1) / 2
    gw_c = gx1 - gx0
    gh_c = gy1 - gy0
    xg0 = gcx - gw_c / 2
    xg1 = gcx + gw_c / 2
    yg0 = gcy - gh_c / 2
    yg1 = gcy + gh_c / 2
    in_boxes = (acx > xg0) & (acx < xg1) & (acy > yg0) & (acy < yg1)
    gw2 = xg1 - xg0
    gh2 = yg1 - yg0
    in_centers = ((acx > gcx - CENTER_RADIUS * gw2)
                  & (acx < gcx + CENTER_RADIUS * gw2)
                  & (acy > gcy - CENTER_RADIUS * gh2)
                  & (acy < gcy + CENTER_RADIUS * gh2))
    in_bc = in_boxes & in_centers

    lab_i = jax.lax.broadcasted_iota(jnp.int32, (C, G), 0)
    onehot = (labels == lab_i).astype(jnp.float32)
    glog = jax.lax.dot_general(
        logits, onehot, (((1,), (0,)), ((), ())),
        precision=jax.lax.Precision.HIGHEST,
        preferred_element_type=jnp.float32)
    p = jax.nn.sigmoid(glog)
    pos = (ALPHA * ((1.0 - p) * (1.0 - p))) * -jnp.log(p + 1e-8)
    neg = ((1.0 - ALPHA) * (p * p)) * -jnp.log((1.0 - p) + 1e-8)
    cost_class = pos - neg

    cost = (BBOX_W * cost_bbox + CLASS_W * cost_class
            + GIOU_W * (-giou) + jnp.where(in_bc, 0.0, 100.0))
    cost_ref[0] = cost

    iou_m = iou
    s = jnp.zeros((1, G), jnp.float32)
    for _ in range(OTA_K):
        mv = jnp.max(iou_m, axis=0, keepdims=True)
        s = s + mv
        rsel = jnp.min(jnp.where(iou_m == mv, rowi, Q), axis=0, keepdims=True)
        iou_m = jnp.where(rowi == rsel, -1.0, iou_m)
    dk = jnp.clip(s.astype(jnp.int32), 1, OTA_K)

    cost_m = cost
    matching = jnp.zeros((Q, G), jnp.float32)
    for k in range(OTA_K):
        mv = jnp.min(cost_m, axis=0, keepdims=True)
        rsel = jnp.min(jnp.where(cost_m == mv, rowi, Q), axis=0, keepdims=True)
        hit = rowi == rsel
        matching = jnp.where(hit & (k < dk), 1.0, matching)
        cost_m = jnp.where(hit, BIG, cost_m)

    rowsum = jnp.sum(matching, axis=1, keepdims=True)
    multiple = rowsum > 1.0
    minr = jnp.min(cost, axis=1, keepdims=True)
    amin_r = jnp.min(jnp.where(cost == minr, coli, G), axis=1, keepdims=True)
    matching = jnp.where(multiple,
                         jnp.where(coli == amin_r, 1.0, 0.0),
                         matching)

    colsum = jnp.sum(matching, axis=0, keepdims=True)
    unmatched = colsum < 1.0
    minc = jnp.min(cost, axis=0, keepdims=True)
    amin_c = jnp.min(jnp.where(cost == minc, rowi, Q), axis=0, keepdims=True)
    matching = jnp.maximum(matching,
                           jnp.where((rowi == amin_c) & unmatched, 1.0, 0.0))
    matching_ref[0] = matching

    mcost = jnp.where(matching > 0.0, cost, BIG)
    mn = jnp.min(mcost, axis=0, keepdims=True)
    qidx = jnp.min(jnp.where(mcost == mn, rowi, Q), axis=0, keepdims=True)
    qidx_ref[0] = qidx.astype(jnp.int32)


def kernel(pred_logits, pred_boxes, gt_boxes, gt_labels, image_size):
    gt_t = jnp.transpose(gt_boxes, (0, 2, 1))
    labels3 = gt_labels.reshape(B, 1, G)
    img3 = image_size.reshape(B, 1, 4)
    out_shape = [jax.ShapeDtypeStruct((B, Q, G), jnp.float32),
                 jax.ShapeDtypeStruct((B, Q, G), jnp.float32),
                 jax.ShapeDtypeStruct((B, 1, G), jnp.int32)]
    in_specs = [pl.BlockSpec((1, Q, C), lambda b: (b, 0, 0)),
                pl.BlockSpec((1, Q, 4), lambda b: (b, 0, 0)),
                pl.BlockSpec((1, 4, G), lambda b: (b, 0, 0)),
                pl.BlockSpec((1, 1, G), lambda b: (b, 0, 0)),
                pl.BlockSpec((1, 1, 4), lambda b: (b, 0, 0))]
    out_specs = [pl.BlockSpec((1, Q, G), lambda b: (b, 0, 0)),
                 pl.BlockSpec((1, Q, G), lambda b: (b, 0, 0)),
                 pl.BlockSpec((1, 1, G), lambda b: (b, 0, 0))]
    cost, matching, qidx = pl.pallas_call(
        _match_kernel,
        grid=(B,),
        in_specs=in_specs,
        out_specs=out_specs,
        out_shape=out_shape,
    )(pred_logits, pred_boxes, gt_t, labels3, img3)
    return cost, matching, qidx.reshape(B, G)


# speedup vs baseline: 13.0821x; 1.0162x over previous
import jax
import jax.numpy as jnp
from jax.experimental import pallas as pl

B, Q, C, G = 16, 4096, 80, 100
OTA_K = 5
ALPHA, GAMMA = 0.25, 2.0
CLASS_W, BBOX_W, GIOU_W = 1.0, 5.0, 2.0
CENTER_RADIUS = 2.5
BIG = 1e30


def _match_kernel(logits_ref, boxes_ref, gt_t_ref, labels_ref, img_ref,
                  cost_ref, matching_ref, qidx_ref):
    logits = logits_ref[0]
    boxes = boxes_ref[0]
    gt_t = gt_t_ref[0]
    labels = labels_ref[0]
    img = img_ref[0]

    rowi = jax.lax.broadcasted_iota(jnp.int32, (Q, G), 0)
    coli = jax.lax.broadcasted_iota(jnp.int32, (Q, G), 1)

    bx0 = boxes[:, 0:1]
    by0 = boxes[:, 1:2]
    bx1 = boxes[:, 2:3]
    by1 = boxes[:, 3:4]
    gx0 = gt_t[0:1, :]
    gy0 = gt_t[1:2, :]
    gx1 = gt_t[2:3, :]
    gy1 = gt_t[3:4, :]

    i0 = img[0:1, 0:1]
    i1 = img[0:1, 1:2]
    i2 = img[0:1, 2:3]
    i3 = img[0:1, 3:4]
    cost_bbox = (jnp.abs(bx0 / i0 - gx0 / i0)
                 + jnp.abs(by0 / i1 - gy0 / i1)
                 + jnp.abs(bx1 / i2 - gx1 / i2)
                 + jnp.abs(by1 / i3 - gy1 / i3))

    area_a = (bx1 - bx0) * (by1 - by0)
    area_b = (gx1 - gx0) * (gy1 - gy0)
    w = jnp.maximum(jnp.minimum(bx1, gx1) - jnp.maximum(bx0, gx0), 0.0)
    h = jnp.maximum(jnp.minimum(by1, gy1) - jnp.maximum(by0, gy0), 0.0)
    inter = w * h
    union = (area_a + area_b) - inter
    iou = inter / (union + 1e-8)
    w2 = jnp.maximum(jnp.maximum(bx1, gx1) - jnp.minimum(bx0, gx0), 0.0)
    h2 = jnp.maximum(jnp.maximum(by1, gy1) - jnp.minimum(by0, gy0), 0.0)
    area_c = w2 * h2
    giou = iou - (area_c - union) / (area_c + 1e-8)

    acx = (bx0 + bx1) / 2
    acy = (by0 + by1) / 2
    gcx = (gx0 + gx1) / 2
    gcy = (gy0 + gy1) / 2
    gw_c = gx1 - gx0
    gh_c = gy1 - gy0
    xg0 = gcx - gw_c / 2
    xg1 = gcx + gw_c / 2
    yg0 = gcy - gh_c / 2
    yg1 = gcy + gh_c / 2
    in_bc = (acx > xg0) & (acx < xg1) & (acy > yg0) & (acy < yg1)

    lab_i = jax.lax.broadcasted_iota(jnp.int32, (C, G), 0)
    onehot = (labels == lab_i).astype(jnp.float32)
    glog = jax.lax.dot_general(
        logits, onehot, (((1,), (0,)), ((), ())),
        precision=jax.lax.Precision.HIGHEST,
        preferred_element_type=jnp.float32)
    p = jax.nn.sigmoid(glog)
    pos = (ALPHA * ((1.0 - p) * (1.0 - p))) * -jnp.log(p + 1e-8)
    neg = ((1.0 - ALPHA) * (p * p)) * -jnp.log((1.0 - p) + 1e-8)
    cost_class = pos - neg

    cost = (BBOX_W * cost_bbox + CLASS_W * cost_class
            + GIOU_W * (-giou) + jnp.where(in_bc, 0.0, 100.0))
    cost_ref[0] = cost

    NCH = 64
    SCH = Q // NCH
    iou3 = iou.reshape(NCH, SCH, G)
    regs = [jnp.full((SCH, G), -1.0, jnp.float32) for _ in range(OTA_K)]
    for c in range(NCH):
        x = iou3[c]
        for k in range(OTA_K):
            t = jnp.maximum(regs[k], x)
            x = jnp.minimum(regs[k], x)
            regs[k] = t
    cand = jnp.concatenate(regs, axis=0)
    crow = jax.lax.broadcasted_iota(jnp.int32, (OTA_K * SCH, G), 0)
    s = jnp.zeros((1, G), jnp.float32)
    for _ in range(OTA_K):
        mv = jnp.max(cand, axis=0, keepdims=True)
        s = s + mv
        rsel = jnp.min(jnp.where(cand == mv, crow, OTA_K * SCH),
                       axis=0, keepdims=True)
        cand = jnp.where(crow == rsel, -1.0, cand)
    dk = jnp.clip(s.astype(jnp.int32), 1, OTA_K)

    kmax = jnp.max(dk)

    def cost_topk_body(k, carry):
        cost_m, matching = carry
        mv = jnp.min(cost_m, axis=0, keepdims=True)
        rsel = jnp.min(jnp.where(cost_m == mv, rowi, Q), axis=0, keepdims=True)
        hit = rowi == rsel
        matching = jnp.where(hit & (k < dk), 1.0, matching)
        cost_m = jnp.where(hit, BIG, cost_m)
        return cost_m, matching

    _, matching = jax.lax.fori_loop(
        0, kmax, cost_topk_body,
        (cost, jnp.zeros((Q, G), jnp.float32)))

    rowsum = jnp.sum(matching, axis=1, keepdims=True)
    multiple = rowsum > 1.0
    minr = jnp.min(cost, axis=1, keepdims=True)
    amin_r = jnp.min(jnp.where(cost == minr, coli, G), axis=1, keepdims=True)
    matching = jnp.where(multiple,
                         jnp.where(coli == amin_r, 1.0, 0.0),
                         matching)

    colsum = jnp.sum(matching, axis=0, keepdims=True)
    unmatched = colsum < 1.0
    minc = jnp.min(cost, axis=0, keepdims=True)
    amin_c = jnp.min(jnp.where(cost == minc, rowi, Q), axis=0, keepdims=True)
    matching = jnp.maximum(matching,
                           jnp.where((rowi == amin_c) & unmatched, 1.0, 0.0))
    matching_ref[0] = matching

    mcost = jnp.where(matching > 0.0, cost, BIG)
    mn = jnp.min(mcost, axis=0, keepdims=True)
    qidx = jnp.min(jnp.where(mcost == mn, rowi, Q), axis=0, keepdims=True)
    qidx_ref[0] = qidx.astype(jnp.int32)


def kernel(pred_logits, pred_boxes, gt_boxes, gt_labels, image_size):
    gt_t = jnp.transpose(gt_boxes, (0, 2, 1))
    labels3 = gt_labels.reshape(B, 1, G)
    img3 = image_size.reshape(B, 1, 4)
    out_shape = [jax.ShapeDtypeStruct((B, Q, G), jnp.float32),
                 jax.ShapeDtypeStruct((B, Q, G), jnp.float32),
                 jax.ShapeDtypeStruct((B, 1, G), jnp.int32)]
    in_specs = [pl.BlockSpec((1, Q, C), lambda b: (b, 0, 0)),
                pl.BlockSpec((1, Q, 4), lambda b: (b, 0, 0)),
                pl.BlockSpec((1, 4, G), lambda b: (b, 0, 0)),
                pl.BlockSpec((1, 1, G), lambda b: (b, 0, 0)),
                pl.BlockSpec((1, 1, 4), lambda b: (b, 0, 0))]
    out_specs = [pl.BlockSpec((1, Q, G), lambda b: (b, 0, 0)),
                 pl.BlockSpec((1, Q, G), lambda b: (b, 0, 0)),
                 pl.BlockSpec((1, 1, G), lambda b: (b, 0, 0))]
    cost, matching, qidx = pl.pallas_call(
        _match_kernel,
        grid=(B,),
        in_specs=in_specs,
        out_specs=out_specs,
        out_shape=out_shape,
    )(pred_logits, pred_boxes, gt_t, labels3, img3)
    return cost, matching, qidx.reshape(B, G)


# speedup vs baseline: 21.3762x; 1.6340x over previous
import jax
import jax.numpy as jnp
from jax.experimental import pallas as pl

B, Q, C, G = 16, 4096, 80, 100
OTA_K = 5
ALPHA, GAMMA = 0.25, 2.0
CLASS_W, BBOX_W, GIOU_W = 1.0, 5.0, 2.0
CENTER_RADIUS = 2.5
BIG = 1e30
NCH = 32
SCH = Q // NCH


def _match_kernel(logits_ref, boxes_ref, gt_ref, labels_ref, img_ref,
                  cost_ref, matching_ref, qidx_ref):
    logits_t = logits_ref[0]
    boxes_t = boxes_ref[0]
    gt4 = gt_ref[0]
    labels = labels_ref[0]
    img = img_ref[0]

    gti = jax.lax.broadcasted_iota(jnp.int32, (G, Q), 0)
    lanei = jax.lax.broadcasted_iota(jnp.int32, (G, Q), 1)

    bx0 = boxes_t[0:1, :]
    by0 = boxes_t[1:2, :]
    bx1 = boxes_t[2:3, :]
    by1 = boxes_t[3:4, :]
    gt = jnp.transpose(gt4)
    gx0 = gt[:, 0:1]
    gy0 = gt[:, 1:2]
    gx1 = gt[:, 2:3]
    gy1 = gt[:, 3:4]

    i0 = img[0:1, 0:1]
    i1 = img[1:2, 0:1]
    i2 = img[2:3, 0:1]
    i3 = img[3:4, 0:1]
    cost_bbox = (jnp.abs(bx0 / i0 - gx0 / i0)
                 + jnp.abs(by0 / i1 - gy0 / i1)
                 + jnp.abs(bx1 / i2 - gx1 / i2)
                 + jnp.abs(by1 / i3 - gy1 / i3))

    area_a = (bx1 - bx0) * (by1 - by0)
    area_b = (gx1 - gx0) * (gy1 - gy0)
    w = jnp.maximum(jnp.minimum(bx1, gx1) - jnp.maximum(bx0, gx0), 0.0)
    h = jnp.maximum(jnp.minimum(by1, gy1) - jnp.maximum(by0, gy0), 0.0)
    inter = w * h
    union = (area_a + area_b) - inter
    iou = inter / (union + 1e-8)
    w2 = jnp.maximum(jnp.maximum(bx1, gx1) - jnp.minimum(bx0, gx0), 0.0)
    h2 = jnp.maximum(jnp.maximum(by1, gy1) - jnp.minimum(by0, gy0), 0.0)
    area_c = w2 * h2
    giou = iou - (area_c - union) / (area_c + 1e-8)

    acx = (bx0 + bx1) / 2
    acy = (by0 + by1) / 2
    gcx = (gx0 + gx1) / 2
    gcy = (gy0 + gy1) / 2
    gw_c = gx1 - gx0
    gh_c = gy1 - gy0
    xg0 = gcx - gw_c / 2
    xg1 = gcx + gw_c / 2
    yg0 = gcy - gh_c / 2
    yg1 = gcy + gh_c / 2
    in_bc = (acx > xg0) & (acx < xg1) & (acy > yg0) & (acy < yg1)

    lab_i = jax.lax.broadcasted_iota(jnp.int32, (G, C), 1)
    onehot = (labels == lab_i).astype(jnp.float32)
    glog = jax.lax.dot_general(
        onehot, logits_t, (((1,), (0,)), ((), ())),
        precision=jax.lax.Precision.HIGHEST,
        preferred_element_type=jnp.float32)
    p = jax.nn.sigmoid(glog)
    pos = (ALPHA * ((1.0 - p) * (1.0 - p))) * -jnp.log(p + 1e-8)
    neg = ((1.0 - ALPHA) * (p * p)) * -jnp.log((1.0 - p) + 1e-8)
    cost_class = pos - neg

    cost = (BBOX_W * cost_bbox + CLASS_W * cost_class
            + GIOU_W * (-giou) + jnp.where(in_bc, 0.0, 100.0))
    cost_ref[0] = cost

    regs = [jnp.full((G, SCH), -1.0, jnp.float32) for _ in range(OTA_K)]
    for c in range(NCH):
        x = iou[:, c * SCH:(c + 1) * SCH]
        for k in range(OTA_K):
            t = jnp.maximum(regs[k], x)
            x = jnp.minimum(regs[k], x)
            regs[k] = t
    cand = jnp.concatenate(regs, axis=1)
    clane = jax.lax.broadcasted_iota(jnp.int32, (G, OTA_K * SCH), 1)
    s = jnp.zeros((G, 1), jnp.float32)
    for _ in range(OTA_K):
        mv = jnp.max(cand, axis=1, keepdims=True)
        s = s + mv
        rsel = jnp.min(jnp.where(cand == mv, clane, OTA_K * SCH),
                       axis=1, keepdims=True)
        cand = jnp.where(clane == rsel, -1.0, cand)
    dk = jnp.clip(s.astype(jnp.int32), 1, OTA_K)

    kmax = jnp.max(dk)

    def cost_topk_body(k, carry):
        cost_m, matching = carry
        mv = jnp.min(cost_m, axis=1, keepdims=True)
        rsel = jnp.min(jnp.where(cost_m == mv, lanei, Q),
                       axis=1, keepdims=True)
        hit = lanei == rsel
        matching = jnp.where(hit & (k < dk), 1.0, matching)
        cost_m = jnp.where(hit, BIG, cost_m)
        return cost_m, matching

    _, matching = jax.lax.fori_loop(
        0, kmax, cost_topk_body,
        (cost, jnp.zeros((G, Q), jnp.float32)))

    qsum = jnp.sum(matching, axis=0, keepdims=True)
    multiple = qsum > 1.0
    minq = jnp.min(cost, axis=0, keepdims=True)
    aminq = jnp.min(jnp.where(cost == minq, gti, G), axis=0, keepdims=True)
    matching = jnp.where(multiple,
                         jnp.where(gti == aminq, 1.0, 0.0),
                         matching)

    gsum = jnp.sum(matching, axis=1, keepdims=True)
    unmatched = gsum < 1.0
    ming = jnp.min(cost, axis=1, keepdims=True)
    aming = jnp.min(jnp.where(cost == ming, lanei, Q), axis=1, keepdims=True)
    matching = jnp.maximum(matching,
                           jnp.where((lanei == aming) & unmatched, 1.0, 0.0))
    matching_ref[0] = matching

    mcost = jnp.where(matching > 0.0, cost, BIG)
    mn = jnp.min(mcost, axis=1, keepdims=True)
    qidx = jnp.min(jnp.where(mcost == mn, lanei, Q), axis=1, keepdims=True)
    qidx_ref[0] = qidx.astype(jnp.int32)


def kernel(pred_logits, pred_boxes, gt_boxes, gt_labels, image_size):
    logits_t = jnp.transpose(pred_logits, (0, 2, 1))
    boxes_t = jnp.transpose(pred_boxes, (0, 2, 1))
    gt_t = jnp.transpose(gt_boxes, (0, 2, 1))
    labels3 = gt_labels.reshape(B, G, 1)
    img3 = image_size.reshape(B, 4, 1)
    out_shape = [jax.ShapeDtypeStruct((B, G, Q), jnp.float32),
                 jax.ShapeDtypeStruct((B, G, Q), jnp.float32),
                 jax.ShapeDtypeStruct((B, G, 1), jnp.int32)]
    in_specs = [pl.BlockSpec((1, C, Q), lambda b: (b, 0, 0)),
                pl.BlockSpec((1, 4, Q), lambda b: (b, 0, 0)),
                pl.BlockSpec((1, 4, G), lambda b: (b, 0, 0)),
                pl.BlockSpec((1, G, 1), lambda b: (b, 0, 0)),
                pl.BlockSpec((1, 4, 1), lambda b: (b, 0, 0))]
    out_specs = [pl.BlockSpec((1, G, Q), lambda b: (b, 0, 0)),
                 pl.BlockSpec((1, G, Q), lambda b: (b, 0, 0)),
                 pl.BlockSpec((1, G, 1), lambda b: (b, 0, 0))]
    cost_t, matching_t, qidx = pl.pallas_call(
        _match_kernel,
        grid=(B,),
        in_specs=in_specs,
        out_specs=out_specs,
        out_shape=out_shape,
    )(logits_t, boxes_t, gt_t, labels3, img3)
    return (jnp.transpose(cost_t, (0, 2, 1)),
            jnp.transpose(matching_t, (0, 2, 1)),
            qidx.reshape(B, G))


# speedup vs baseline: 21.6929x; 1.0148x over previous
import jax
import jax.numpy as jnp
from jax.experimental import pallas as pl

B, Q, C, G = 16, 4096, 80, 100
OTA_K = 5
ALPHA, GAMMA = 0.25, 2.0
CLASS_W, BBOX_W, GIOU_W = 1.0, 5.0, 2.0
CENTER_RADIUS = 2.5
BIG = 1e30
NCH = 32
SCH = Q // NCH


def _match_kernel(logits_ref, boxes_ref, gt_ref, labels_ref, img_ref,
                  cost_ref, matching_ref, qidx_ref):
    logits_t = logits_ref[0]
    boxes_t = boxes_ref[0]
    gt4 = gt_ref[0]
    labels = labels_ref[0]
    img = img_ref[0]

    gti = jax.lax.broadcasted_iota(jnp.int32, (G, Q), 0)
    lanei = jax.lax.broadcasted_iota(jnp.int32, (G, Q), 1)

    bx0 = boxes_t[0:1, :]
    by0 = boxes_t[1:2, :]
    bx1 = boxes_t[2:3, :]
    by1 = boxes_t[3:4, :]
    gt = jnp.transpose(gt4)
    gx0 = gt[:, 0:1]
    gy0 = gt[:, 1:2]
    gx1 = gt[:, 2:3]
    gy1 = gt[:, 3:4]

    i0 = img[0:1, 0:1]
    i1 = img[1:2, 0:1]
    i2 = img[2:3, 0:1]
    i3 = img[3:4, 0:1]
    cost_bbox = (jnp.abs(bx0 / i0 - gx0 / i0)
                 + jnp.abs(by0 / i1 - gy0 / i1)
                 + jnp.abs(bx1 / i2 - gx1 / i2)
                 + jnp.abs(by1 / i3 - gy1 / i3))

    area_a = (bx1 - bx0) * (by1 - by0)
    area_b = (gx1 - gx0) * (gy1 - gy0)
    w = jnp.maximum(jnp.minimum(bx1, gx1) - jnp.maximum(bx0, gx0), 0.0)
    h = jnp.maximum(jnp.minimum(by1, gy1) - jnp.maximum(by0, gy0), 0.0)
    inter = w * h
    union = (area_a + area_b) - inter
    iou = inter / (union + 1e-8)
    w2 = jnp.maximum(jnp.maximum(bx1, gx1) - jnp.minimum(bx0, gx0), 0.0)
    h2 = jnp.maximum(jnp.maximum(by1, gy1) - jnp.minimum(by0, gy0), 0.0)
    area_c = w2 * h2
    giou = iou - (area_c - union) / (area_c + 1e-8)

    acx = (bx0 + bx1) / 2
    acy = (by0 + by1) / 2
    gcx = (gx0 + gx1) / 2
    gcy = (gy0 + gy1) / 2
    gw_c = gx1 - gx0
    gh_c = gy1 - gy0
    xg0 = gcx - gw_c / 2
    xg1 = gcx + gw_c / 2
    yg0 = gcy - gh_c / 2
    yg1 = gcy + gh_c / 2
    in_bc = (acx > xg0) & (acx < xg1) & (acy > yg0) & (acy < yg1)

    lab_i = jax.lax.broadcasted_iota(jnp.int32, (G, C), 1)
    onehot = (labels == lab_i).astype(jnp.float32)
    glog = jax.lax.dot_general(
        onehot, logits_t, (((1,), (0,)), ((), ())),
        precision=jax.lax.Precision.HIGHEST,
        preferred_element_type=jnp.float32)
    p = jax.nn.sigmoid(glog)
    pos = (ALPHA * ((1.0 - p) * (1.0 - p))) * -jnp.log(p + 1e-8)
    neg = ((1.0 - ALPHA) * (p * p)) * -jnp.log((1.0 - p) + 1e-8)
    cost_class = pos - neg

    cost = (BBOX_W * cost_bbox + CLASS_W * cost_class
            + GIOU_W * (-giou) + jnp.where(in_bc, 0.0, 100.0))
    cost_ref[0] = jnp.transpose(cost)

    regs = [jnp.full((G, SCH), -1.0, jnp.float32) for _ in range(OTA_K)]
    for c in range(NCH):
        x = iou[:, c * SCH:(c + 1) * SCH]
        for k in range(OTA_K):
            t = jnp.maximum(regs[k], x)
            x = jnp.minimum(regs[k], x)
            regs[k] = t
    cand = jnp.concatenate(regs, axis=1)
    clane = jax.lax.broadcasted_iota(jnp.int32, (G, OTA_K * SCH), 1)
    s = jnp.zeros((G, 1), jnp.float32)
    for _ in range(OTA_K):
        mv = jnp.max(cand, axis=1, keepdims=True)
        s = s + mv
        rsel = jnp.min(jnp.where(cand == mv, clane, OTA_K * SCH),
                       axis=1, keepdims=True)
        cand = jnp.where(clane == rsel, -1.0, cand)
    dk = jnp.clip(s.astype(jnp.int32), 1, OTA_K)

    kmax = jnp.max(dk)

    def cost_topk_body(k, carry):
        cost_m, matching = carry
        mv = jnp.min(cost_m, axis=1, keepdims=True)
        rsel = jnp.min(jnp.where(cost_m == mv, lanei, Q),
                       axis=1, keepdims=True)
        hit = lanei == rsel
        matching = jnp.where(hit & (k < dk), 1.0, matching)
        cost_m = jnp.where(hit, BIG, cost_m)
        return cost_m, matching

    _, matching = jax.lax.fori_loop(
        0, kmax, cost_topk_body,
        (cost, jnp.zeros((G, Q), jnp.float32)))

    qsum = jnp.sum(matching, axis=0, keepdims=True)
    multiple = qsum > 1.0
    minq = jnp.min(cost, axis=0, keepdims=True)
    aminq = jnp.min(jnp.where(cost == minq, gti, G), axis=0, keepdims=True)
    matching = jnp.where(multiple,
                         jnp.where(gti == aminq, 1.0, 0.0),
                         matching)

    gsum = jnp.sum(matching, axis=1, keepdims=True)
    unmatched = gsum < 1.0
    ming = jnp.min(cost, axis=1, keepdims=True)
    aming = jnp.min(jnp.where(cost == ming, lanei, Q), axis=1, keepdims=True)
    matching = jnp.maximum(matching,
                           jnp.where((lanei == aming) & unmatched, 1.0, 0.0))
    matching_ref[0] = jnp.transpose(matching)

    mcost = jnp.where(matching > 0.0, cost, BIG)
    mn = jnp.min(mcost, axis=1, keepdims=True)
    qidx = jnp.min(jnp.where(mcost == mn, lanei, Q), axis=1, keepdims=True)
    qidx_ref[0] = qidx.astype(jnp.int32)


def kernel(pred_logits, pred_boxes, gt_boxes, gt_labels, image_size):
    logits_t = jnp.transpose(pred_logits, (0, 2, 1))
    boxes_t = jnp.transpose(pred_boxes, (0, 2, 1))
    gt_t = jnp.transpose(gt_boxes, (0, 2, 1))
    labels3 = gt_labels.reshape(B, G, 1)
    img3 = image_size.reshape(B, 4, 1)
    out_shape = [jax.ShapeDtypeStruct((B, Q, G), jnp.float32),
                 jax.ShapeDtypeStruct((B, Q, G), jnp.float32),
                 jax.ShapeDtypeStruct((B, G, 1), jnp.int32)]
    in_specs = [pl.BlockSpec((1, C, Q), lambda b: (b, 0, 0)),
                pl.BlockSpec((1, 4, Q), lambda b: (b, 0, 0)),
                pl.BlockSpec((1, 4, G), lambda b: (b, 0, 0)),
                pl.BlockSpec((1, G, 1), lambda b: (b, 0, 0)),
                pl.BlockSpec((1, 4, 1), lambda b: (b, 0, 0))]
    out_specs = [pl.BlockSpec((1, Q, G), lambda b: (b, 0, 0)),
                 pl.BlockSpec((1, Q, G), lambda b: (b, 0, 0)),
                 pl.BlockSpec((1, G, 1), lambda b: (b, 0, 0))]
    cost, matching, qidx = pl.pallas_call(
        _match_kernel,
        grid=(B,),
        in_specs=in_specs,
        out_specs=out_specs,
        out_shape=out_shape,
    )(logits_t, boxes_t, gt_t, labels3, img3)
    return cost, matching, qidx.reshape(B, G)
